# baseline (speedup 1.0000x reference)
import functools

import jax
import jax.numpy as jnp
from jax import lax
from jax.experimental import pallas as pl
from jax.experimental.pallas import tpu as pltpu
from jax.experimental.pallas import tpu_sc as plsc

NEIGHBORS, TIMESTEPS = 16, 8
_NW = 32


def _knn_body(qpts_ref, kpts_ref, idx_ref, rel_ref, *, K, D, N, want_rel):
    d2 = None
    diff0 = None
    for d in range(D):
        qcol = qpts_ref[0, :, d:d + 1]
        krow = kpts_ref[0, d:d + 1, :]
        diff = qcol - krow
        sq = diff * diff
        d2 = sq if d2 is None else d2 + sq
        if want_rel and d == 0:
            diff0 = diff
    iota = jax.lax.broadcasted_iota(jnp.int32, (1, N), 1)
    big = jnp.int32(N)
    for k in range(K):
        if want_rel:
            m = jnp.min(d2, axis=1, keepdims=True)
            am = jnp.min(jnp.where(d2 == m, iota, big), axis=1,
                         keepdims=True)
        else:
            am = jnp.argmin(d2, axis=1).astype(jnp.int32)[:, None]
        idx_ref[0, :, k:k + 1] = am
        hit = iota == am
        if want_rel:
            rel_ref[0, :, k:k + 1] = jnp.sum(
                jnp.where(hit, diff0, 0.0), axis=1, keepdims=True)
        d2 = jnp.where(hit, jnp.inf, d2)


def _knn(qpts, kpts_t, K, want_rel, tq):
    B, Nq, D = qpts.shape
    N = kpts_t.shape[2]
    grid = (B, Nq // tq)
    out_shapes = [jax.ShapeDtypeStruct((B, Nq, K), jnp.int32)]
    out_specs = [pl.BlockSpec((1, tq, K), lambda b, i: (b, i, 0))]
    if want_rel:
        out_shapes.append(jax.ShapeDtypeStruct((B, Nq, K), jnp.float32))
        out_specs.append(pl.BlockSpec((1, tq, K), lambda b, i: (b, i, 0)))
    body = functools.partial(_knn_body, K=K, D=D, N=N, want_rel=want_rel)
    if not want_rel:
        body2 = lambda q, kk, i: body(q, kk, i, None)
    else:
        body2 = body
    res = pl.pallas_call(
        body2,
        grid=grid,
        in_specs=[
            pl.BlockSpec((1, tq, D), lambda b, i: (b, i, 0)),
            pl.BlockSpec((1, D, N), lambda b, i: (b, 0, 0)),
        ],
        out_specs=out_specs,
        out_shape=out_shapes,
    )(qpts, kpts_t)
    return res if want_rel else (res[0], None)


_NBUF = 4


@functools.lru_cache(maxsize=None)
def _gmean_sc_v2(NQ, K, C, per_qk, CQ):
    nq_w = NQ // _NW
    steps = nq_w // CQ
    assert CQ * K <= 128 and steps % _NBUF == 0
    nj = C // 16
    nk_rows = CQ * K
    mesh = plsc.VectorSubcoreMesh(core_axis_name="c", subcore_axis_name="s")
    add_rows = nk_rows if per_qk else nq_w

    @functools.partial(
        pl.kernel, mesh=mesh,
        out_type=jax.ShapeDtypeStruct((NQ, C), jnp.float32),
        scratch_types=(
            [pltpu.VMEM((nq_w * K,), jnp.int32)]
            + [pltpu.VMEM((nk_rows, 128), jnp.float32)] * _NBUF
            + [pltpu.VMEM((add_rows, C), jnp.float32)]
            + [pltpu.VMEM((nk_rows if per_qk else 1, C), jnp.float32)] * (_NBUF - 1)
            + [pltpu.VMEM((nq_w, C), jnp.float32)]
            + [pltpu.SemaphoreType.DMA] * _NBUF
        ))
    def kfun(z_hbm, gidx_hbm, add_hbm, out_hbm, idx_all, *sc):
        rows_b = sc[0:_NBUF]
        add_b = sc[_NBUF:2 * _NBUF]
        out_v = sc[2 * _NBUF]
        sem_b = sc[2 * _NBUF + 1:]
        wid = lax.axis_index("s") * 2 + lax.axis_index("c")
        qbase = wid * nq_w

        pltpu.sync_copy(gidx_hbm.at[pl.ds(qbase * K, nq_w * K)], idx_all)
        if not per_qk:
            pltpu.sync_copy(add_hbm.at[pl.ds(qbase, nq_w)], add_b[0])

        def fire(c, b):
            pltpu.async_copy(
                z_hbm.at[idx_all.at[pl.ds(c * nk_rows, nk_rows)]],
                rows_b[b], sem_b[b])
            if per_qk:
                pltpu.async_copy(
                    add_hbm.at[pl.ds((qbase + c * CQ) * K, nk_rows)],
                    add_b[b], sem_b[b])

        def drain(b):
            pltpu.make_async_copy(
                z_hbm.at[pl.ds(0, nk_rows)], rows_b[b], sem_b[b]).wait()
            if per_qk:
                pltpu.make_async_copy(
                    add_hbm.at[pl.ds(0, nk_rows)], add_b[b], sem_b[b]).wait()

        def compute(c, b):
            rows = rows_b[b]
            adds = add_b[b]
            qloc = c * CQ
            inv_k = jnp.float32(1.0 / K)
            for q in range(CQ):
                for j in range(nj):
                    sl = pl.ds(j * 16, 16)
                    acc = jnp.zeros((16,), jnp.float32)
                    if not per_qk:
                        a = add_b[0][qloc + q, sl]
                    for k in range(K):
                        r = q * K + k
                        h = rows[r, sl] + (adds[r, sl] if per_qk else a)
                        acc = acc + jnp.maximum(h, 0.0)
                    out_v[qloc + q, sl] = acc * inv_k

        for b in range(_NBUF):
            fire(jnp.int32(b), b)

        def step(s, carry):
            c0 = _NBUF * s
            for b in range(_NBUF):
                drain(b)
                compute(c0 + b, b)
                fire(jnp.minimum(c0 + b + _NBUF, steps - 1), b)
            return carry

        lax.fori_loop(0, steps // _NBUF, step, 0)
        for b in range(_NBUF):
            drain(b)
        pltpu.sync_copy(out_v, out_hbm.at[pl.ds(qbase, nq_w)])

    return kfun


def _gmean(z, idx, add_pq=None, add_pqk=None):
    B, NR, C = z.shape
    _, NQ_b, K = idx.shape
    NQ = B * NQ_b
    gidx = (idx + (jnp.arange(B, dtype=jnp.int32) * NR)[:, None, None])
    gidx = gidx.reshape(NQ * K)
    zf = z.reshape(B * NR, C)
    if C < 128:
        zf = jnp.pad(zf, ((0, 0), (0, 128 - C)))
    if add_pqk is not None:
        add = add_pqk.reshape(NQ * K, C)
        per_qk = True
    else:
        add = add_pq.reshape(NQ, C)
        per_qk = False
    CQ = (64 if C <= 64 else 32) // K
    out = _gmean_sc_v2(NQ, K, C, per_qk, CQ)(zf, gidx, add)
    return out.reshape(B, NQ_b, C)


def _row_spec(tr, cols):
    return pl.BlockSpec((tr, cols), lambda i: (i, 0))


def _full_spec(shape):
    return pl.BlockSpec(shape, lambda i: tuple(0 for _ in shape))


def _dense_call(body, ins, out_cols, tr=512):
    R = ins[0].shape[0]
    in_specs = [_row_spec(tr, ins[0].shape[1])]
    for a in ins[1:]:
        in_specs.append(_full_spec(a.shape))
    return pl.pallas_call(
        body,
        grid=(R // tr,),
        in_specs=in_specs,
        out_specs=_row_spec(tr, out_cols),
        out_shape=jax.ShapeDtypeStruct((R, out_cols), jnp.float32),
    )(*ins)


def _dot(a, b):
    return jnp.dot(a, b, preferred_element_type=jnp.float32)


def _zs_as(x, sp, sWf, sWr, sb):
    def body(x_ref, sp_ref, wf_ref, wr_ref, b_ref, zs_ref, as_ref):
        u = _dot(sp_ref[...], wr_ref[...])
        zs_ref[...] = _dot(x_ref[...], wf_ref[...]) - u
        as_ref[...] = u + b_ref[...]

    R = x.shape[0]
    tr = 512
    no = sWf.shape[1]
    return pl.pallas_call(
        body,
        grid=(R // tr,),
        in_specs=[_row_spec(tr, x.shape[1]), _row_spec(tr, sp.shape[1]),
                  _full_spec(sWf.shape), _full_spec(sWr.shape),
                  _full_spec(sb.shape)],
        out_specs=[_row_spec(tr, no), _row_spec(tr, no)],
        out_shape=[jax.ShapeDtypeStruct((R, no), jnp.float32)] * 2,
    )(x, sp, sWf, sWr, sb)


def _enc_proj(rel_f, te_w, te_ph, Wr, b):
    def body(r_ref, w_ref, p_ref, wr_ref, b_ref, o_ref):
        e = jnp.cos(r_ref[...] * w_ref[...] + p_ref[...])
        o_ref[...] = _dot(e, wr_ref[...]) + b_ref[...]

    return _dense_call(body, (rel_f, te_w, te_ph, Wr, b), Wr.shape[1],
                       tr=4096)


def _sum2_proj(x, y, Wx, Wy):
    def body(x_ref, y_ref, wx_ref, wy_ref, o_ref):
        o_ref[...] = _dot(x_ref[...], wx_ref[...]) + _dot(y_ref[...],
                                                          wy_ref[...])

    R = x.shape[0]
    tr = 512
    return pl.pallas_call(
        body,
        grid=(R // tr,),
        in_specs=[_row_spec(tr, x.shape[1]), _row_spec(tr, y.shape[1]),
                  _full_spec(Wx.shape), _full_spec(Wy.shape)],
        out_specs=_row_spec(tr, Wx.shape[1]),
        out_shape=jax.ShapeDtypeStruct((R, Wx.shape[1]), jnp.float32),
    )(x, y, Wx, Wy)


def _mlp3(x, sn, tn, W1a, W1b, W1c, b1, W2, b2):
    def body(x_ref, s_ref, t_ref, wa_ref, wb_ref, wc_ref, b1_ref,
             w2_ref, b2_ref, o_ref):
        h = jnp.maximum(
            _dot(x_ref[...], wa_ref[...]) + _dot(s_ref[...], wb_ref[...])
            + _dot(t_ref[...], wc_ref[...]) + b1_ref[...], 0.0)
        o_ref[...] = _dot(h, w2_ref[...]) + b2_ref[...]

    R = x.shape[0]
    tr = 512
    return pl.pallas_call(
        body,
        grid=(R // tr,),
        in_specs=[_row_spec(tr, x.shape[1]), _row_spec(tr, sn.shape[1]),
                  _row_spec(tr, tn.shape[1]), _full_spec(W1a.shape),
                  _full_spec(W1b.shape), _full_spec(W1c.shape),
                  _full_spec(b1.shape), _full_spec(W2.shape),
                  _full_spec(b2.shape)],
        out_specs=_row_spec(tr, W2.shape[1]),
        out_shape=jax.ShapeDtypeStruct((R, W2.shape[1]), jnp.float32),
    )(x, sn, tn, W1a, W1b, W1c, b1, W2, b2)


def _proj(x, W):
    def body(x_ref, w_ref, o_ref):
        o_ref[...] = _dot(x_ref[...], w_ref[...])

    return _dense_call(body, (x, W), W.shape[1])


def kernel(data, ids, space_pts, time_pts, query_pts, te_w, te_phase,
           s0_Wf, s0_Wr, s0_b, t0_Wf, t0_Wr, t0_b, c0_W1, c0_b1, c0_W2, c0_b2,
           s1_Wf, s1_Wr, s1_b, t1_Wf, t1_Wr, t1_b, c1_W1, c1_b1, c1_W2, c1_b2,
           tg_Wf, tg_Wr, tg_b):
    B, N, F = data.shape
    Q = query_pts.shape[1]
    R = B * N

    sp_t = jnp.transpose(space_pts, (0, 2, 1))
    tp_t = jnp.transpose(time_pts, (0, 2, 1))

    idx_s, _ = _knn(space_pts, sp_t, NEIGHBORS, False, 256)
    idx_t, rel_t = _knn(time_pts, tp_t, TIMESTEPS, True, 256)
    idx_q, rel_q = _knn(query_pts, tp_t, TIMESTEPS, True, 512)

    xf = data.reshape(R, F)
    spf = space_pts.reshape(R, 3)
    rel_tf = rel_t.reshape(R * TIMESTEPS, 1)
    rel_qf = rel_q.reshape(B * Q * TIMESTEPS, 1)
    wrow = te_w.reshape(1, -1)
    prow = te_phase.reshape(1, -1)

    x = xf
    for (sWf, sWr, sb, tWf, tWr, tb, cW1, cb1, cW2, cb2) in [
            (s0_Wf, s0_Wr, s0_b, t0_Wf, t0_Wr, t0_b, c0_W1, c0_b1, c0_W2, c0_b2),
            (s1_Wf, s1_Wr, s1_b, t1_Wf, t1_Wr, t1_b, c1_W1, c1_b1, c1_W2, c1_b2)]:
        z_s, a_s = _zs_as(x, spf, sWf, sWr, sb.reshape(1, -1))
        snei = _gmean(z_s.reshape(B, N, -1), idx_s,
                      add_pq=a_s).reshape(R, -1)
        z_t = _sum2_proj(x, snei, tWf[:F], tWf[F:])
        relc = _enc_proj(rel_tf, wrow, prow, tWr, tb.reshape(1, -1))
        tnei = _gmean(z_t.reshape(B, N, -1), idx_t,
                      add_pqk=relc.reshape(B, N, TIMESTEPS, -1)).reshape(R, -1)
        x = _mlp3(x, snei, tnei, cW1[:F], cW1[F:F + 64], cW1[F + 64:],
                  cb1.reshape(1, -1), cW2, cb2.reshape(1, -1))
    z_g = _proj(x, tg_Wf)
    relc_q = _enc_proj(rel_qf, wrow, prow, tg_Wr, tg_b.reshape(1, -1))
    return _gmean(z_g.reshape(B, N, -1), idx_q,
                  add_pqk=relc_q.reshape(B, Q, TIMESTEPS, -1))
